# baseline (speedup 1.0000x reference)
import jax
import jax.numpy as jnp
from jax.experimental import pallas as pl

_B, _C, _S = 256, 768, 196
_NC = 10
_SB = 7
_NSTEP = _S // _SB


def _body(f_ref, w_ref, b_ref, o_ref):
    i = pl.program_id(0)
    partial = jnp.sum(f_ref[...], axis=0) * (1.0 / _S)
    contrib = jax.lax.dot_general(
        partial, w_ref[...], (((1,), (1,)), ((), ())),
        preferred_element_type=jnp.float32)

    @pl.when(i == 0)
    def _init():
        o_ref[...] = contrib + b_ref[...]

    @pl.when(i > 0)
    def _acc():
        o_ref[...] += contrib


def kernel(features, W, b):
    f3 = features.transpose(2, 3, 0, 1).reshape(_S, _B, _C)
    out = pl.pallas_call(
        _body,
        grid=(_NSTEP,),
        in_specs=[
            pl.BlockSpec((_SB, _B, _C), lambda i: (i, 0, 0)),
            pl.BlockSpec((_NC, _C), lambda i: (0, 0)),
            pl.BlockSpec((1, _NC), lambda i: (0, 0)),
        ],
        out_specs=pl.BlockSpec((_B, _NC), lambda i: (0, 0)),
        out_shape=jax.ShapeDtypeStruct((_B, _NC), jnp.float32),
    )(f3, W, b.reshape(1, _NC))
    return out
